# baseline (speedup 1.0000x reference)
import functools

import jax
import jax.numpy as jnp
import numpy as np
from jax.experimental import pallas as pl
from jax.experimental.pallas import tpu as pltpu

TOP_K = 4
N_EXPERTS = 64
BALANCE_COEFF = 0.01


def _hexagram_consts():
    combos = np.zeros((64, 8), dtype=np.float32)
    for e in range(64):
        for b in range(6):
            combos[e, b] = float((e >> (5 - b)) & 1)
    return combos


_HEX_NP = _hexagram_consts()


def _moe_kernel(n_total, x_ref, wrt_ref, temp_ref, hexT_ref, hexsq_ref,
                w1_ref, w2_ref,
                out_ref, loss_ref,
                a_ref, b_ref, psum_ref):
    t = pl.program_id(0)
    u = pl.program_id(1)
    n_t = pl.num_programs(0)

    @pl.when(u == 0)
    def _router():
        x = x_ref[...]
        z = jnp.dot(x, wrt_ref[...], preferred_element_type=jnp.float32)
        zsq = jnp.sum(z * z, axis=1, keepdims=True)
        cross = jnp.dot(z, hexT_ref[...], preferred_element_type=jnp.float32,
                        precision=jax.lax.Precision.HIGHEST)
        d2 = jnp.maximum(zsq - 2.0 * cross + hexsq_ref[...], 0.0)
        temp = temp_ref[0, 0]
        logits = -jnp.sqrt(d2) / temp

        n = logits.shape[0]
        iota_e = jax.lax.broadcasted_iota(jnp.int32, (n, N_EXPERTS), 1)
        cur = logits
        vals = []
        idxs = []
        for _ in range(TOP_K):
            v = jnp.max(cur, axis=1, keepdims=True)
            is_max = cur == v
            idx = jnp.min(jnp.where(is_max, iota_e, N_EXPERTS), axis=1,
                          keepdims=True)
            cur = jnp.where(iota_e == idx, -jnp.inf, cur)
            vals.append(v)
            idxs.append(idx)

        exps = [jnp.exp(v - vals[0]) for v in vals]
        denom = exps[0] + exps[1] + exps[2] + exps[3]

        iota8 = jax.lax.broadcasted_iota(jnp.int32, (n, 8), 1)
        a = jnp.zeros((n, 8), dtype=jnp.float32)
        b = jnp.zeros((n, 8), dtype=jnp.float32)
        for k in range(TOP_K):
            w = exps[k] / denom
            uid = idxs[k] // 8
            lid = idxs[k] - uid * 8
            a = a + jnp.where(iota8 == uid, w, 0.0)
            b = b + jnp.where(iota8 == lid, w, 0.0)
        a_ref[...] = a
        b_ref[...] = b

        m = jnp.max(logits, axis=1, keepdims=True)
        e = jnp.exp(logits - m)
        p = e / jnp.sum(e, axis=1, keepdims=True)
        psum = jnp.sum(p, axis=0, keepdims=True)

        @pl.when(t == 0)
        def _():
            psum_ref[...] = psum

        @pl.when(t != 0)
        def _():
            psum_ref[...] = psum_ref[...] + psum

        @pl.when(t == n_t - 1)
        def _():
            probs = psum_ref[...] * (1.0 / n_total)
            uniform = 1.0 / N_EXPERTS
            kl = jnp.sum(uniform * (jnp.log(uniform) - jnp.log(probs)))
            loss_ref[0, 0] = BALANCE_COEFF * kl

    x = x_ref[...]
    sel = jax.lax.broadcasted_iota(jnp.int32, a_ref.shape, 1) == u
    aw = jnp.sum(jnp.where(sel, a_ref[...], 0.0), axis=1, keepdims=True)
    bw = jnp.sum(jnp.where(sel, b_ref[...], 0.0), axis=1, keepdims=True)

    h = jnp.dot(x, w1_ref[0], preferred_element_type=jnp.float32)
    half = jax.lax.broadcasted_iota(jnp.int32, h.shape, 1) < (h.shape[1] // 2)
    scale = jnp.where(half, aw, bw)
    h = h * jax.nn.sigmoid(h) * scale
    acc = jnp.dot(h, w2_ref[0], preferred_element_type=jnp.float32)

    @pl.when(u == 0)
    def _init():
        out_ref[...] = acc

    @pl.when(u != 0)
    def _acc():
        out_ref[...] = out_ref[...] + acc


@functools.partial(jax.jit, static_argnames=())
def kernel(x, W_router, log_temp, upper_w1, upper_w2, lower_w1, lower_w2):
    B, T, D = x.shape
    N = B * T
    x_flat = x.reshape(N, D)
    H = upper_w1.shape[-1]
    NB = 1024
    n_t = N // NB

    wrt = jnp.zeros((D, 8), jnp.float32).at[:, :6].set(W_router.T)
    temp = jnp.clip(jnp.exp(log_temp), 0.01, 5.0).reshape(1, 1)
    hexT = jnp.asarray(_HEX_NP.T)
    hexsq = jnp.sum(jnp.asarray(_HEX_NP) ** 2, axis=1)[None, :]

    w1c = jnp.concatenate([upper_w1, lower_w1], axis=2)
    w2c = jnp.concatenate([upper_w2, lower_w2], axis=1)
    grid = (n_t, 8)
    out, loss = pl.pallas_call(
        functools.partial(_moe_kernel, N),
        grid=grid,
        in_specs=[
            pl.BlockSpec((NB, D), lambda t, u: (t, 0)),
            pl.BlockSpec((D, 8), lambda t, u: (0, 0)),
            pl.BlockSpec(memory_space=pltpu.SMEM),
            pl.BlockSpec((8, 64), lambda t, u: (0, 0)),
            pl.BlockSpec((1, 64), lambda t, u: (0, 0)),
            pl.BlockSpec((1, D, 2 * H), lambda t, u: (u, 0, 0)),
            pl.BlockSpec((1, 2 * H, D), lambda t, u: (u, 0, 0)),
        ],
        out_specs=[
            pl.BlockSpec((NB, D), lambda t, u: (t, 0)),
            pl.BlockSpec(memory_space=pltpu.SMEM),
        ],
        out_shape=[
            jax.ShapeDtypeStruct((N, D), jnp.float32),
            jax.ShapeDtypeStruct((1, 1), jnp.float32),
        ],
        scratch_shapes=[
            pltpu.VMEM((NB, 8), jnp.float32),
            pltpu.VMEM((NB, 8), jnp.float32),
            pltpu.VMEM((1, 64), jnp.float32),
        ],
        compiler_params=pltpu.CompilerParams(
            dimension_semantics=("arbitrary", "arbitrary"),
        ),
    )(x_flat, wrt, temp, hexT, hexsq, w1c, w2c)

    return out.reshape(B, T, D), loss.reshape(())


# speedup vs baseline: 1.3736x; 1.3736x over previous
import functools

import jax
import jax.numpy as jnp
import numpy as np
from jax.experimental import pallas as pl
from jax.experimental.pallas import tpu as pltpu

TOP_K = 4
N_EXPERTS = 64
BALANCE_COEFF = 0.01


def _hexagram_consts():
    combos = np.zeros((64, 8), dtype=np.float32)
    for e in range(64):
        for b in range(6):
            combos[e, b] = float((e >> (5 - b)) & 1)
    return combos


_HEX_NP = _hexagram_consts()


def _moe_kernel(n_total, x_ref, wrt_ref, temp_ref, hexT_ref, hexsq_ref,
                uw1_ref, uw2_ref, lw1_ref, lw2_ref,
                out_ref, loss_ref,
                a_ref, b_ref, psum_ref):
    t = pl.program_id(0)
    u = pl.program_id(1)
    n_t = pl.num_programs(0)

    @pl.when(u == 0)
    def _router():
        x = x_ref[...]
        z = jnp.dot(x, wrt_ref[...], preferred_element_type=jnp.float32)
        zsq = jnp.sum(z * z, axis=1, keepdims=True)
        cross = jnp.dot(z, hexT_ref[...], preferred_element_type=jnp.float32,
                        precision=jax.lax.Precision.HIGHEST)
        d2 = jnp.maximum(zsq - 2.0 * cross + hexsq_ref[...], 0.0)
        temp = temp_ref[0, 0]
        logits = -jnp.sqrt(d2) / temp

        n = logits.shape[0]
        iota_e = jax.lax.broadcasted_iota(jnp.int32, (n, N_EXPERTS), 1)
        cur = logits
        vals = []
        idxs = []
        for _ in range(TOP_K):
            v = jnp.max(cur, axis=1, keepdims=True)
            is_max = cur == v
            idx = jnp.min(jnp.where(is_max, iota_e, N_EXPERTS), axis=1,
                          keepdims=True)
            cur = jnp.where(iota_e == idx, -jnp.inf, cur)
            vals.append(v)
            idxs.append(idx)

        exps = [jnp.exp(v - vals[0]) for v in vals]
        denom = exps[0] + exps[1] + exps[2] + exps[3]

        iota8 = jax.lax.broadcasted_iota(jnp.int32, (n, 8), 1)
        a = jnp.zeros((n, 8), dtype=jnp.float32)
        b = jnp.zeros((n, 8), dtype=jnp.float32)
        for k in range(TOP_K):
            w = exps[k] / denom
            uid = idxs[k] // 8
            lid = idxs[k] - uid * 8
            a = a + jnp.where(iota8 == uid, w, 0.0)
            b = b + jnp.where(iota8 == lid, w, 0.0)
        a_ref[...] = a
        b_ref[...] = b

        m = jnp.max(logits, axis=1, keepdims=True)
        e = jnp.exp(logits - m)
        p = e / jnp.sum(e, axis=1, keepdims=True)
        psum = jnp.sum(p, axis=0, keepdims=True)

        @pl.when(t == 0)
        def _():
            psum_ref[...] = psum

        @pl.when(t != 0)
        def _():
            psum_ref[...] = psum_ref[...] + psum

        @pl.when(t == n_t - 1)
        def _():
            probs = psum_ref[...] * (1.0 / n_total)
            uniform = 1.0 / N_EXPERTS
            kl = jnp.sum(uniform * (jnp.log(uniform) - jnp.log(probs)))
            loss_ref[0, 0] = BALANCE_COEFF * kl

    x = x_ref[...]
    sel = jax.lax.broadcasted_iota(jnp.int32, a_ref.shape, 1) == u
    aw = jnp.sum(jnp.where(sel, a_ref[...], 0.0), axis=1, keepdims=True)
    bw = jnp.sum(jnp.where(sel, b_ref[...], 0.0), axis=1, keepdims=True)

    h_u = jnp.dot(x, uw1_ref[0], preferred_element_type=jnp.float32)
    h_u = h_u * jax.nn.sigmoid(h_u) * aw
    acc = jnp.dot(h_u, uw2_ref[0], preferred_element_type=jnp.float32)

    h_l = jnp.dot(x, lw1_ref[0], preferred_element_type=jnp.float32)
    h_l = h_l * jax.nn.sigmoid(h_l) * bw
    acc = acc + jnp.dot(h_l, lw2_ref[0], preferred_element_type=jnp.float32)

    @pl.when(u == 0)
    def _init():
        out_ref[...] = acc

    @pl.when(u != 0)
    def _acc():
        out_ref[...] = out_ref[...] + acc


@functools.partial(jax.jit, static_argnames=())
def kernel(x, W_router, log_temp, upper_w1, upper_w2, lower_w1, lower_w2):
    B, T, D = x.shape
    N = B * T
    x_flat = x.reshape(N, D)
    H = upper_w1.shape[-1]
    NB = 2048
    n_t = N // NB

    wrt = jnp.zeros((D, 8), jnp.float32).at[:, :6].set(W_router.T)
    temp = jnp.clip(jnp.exp(log_temp), 0.01, 5.0).reshape(1, 1)
    hexT = jnp.asarray(_HEX_NP.T)
    hexsq = jnp.sum(jnp.asarray(_HEX_NP) ** 2, axis=1)[None, :]

    grid = (n_t, 8)
    out, loss = pl.pallas_call(
        functools.partial(_moe_kernel, N),
        grid=grid,
        in_specs=[
            pl.BlockSpec((NB, D), lambda t, u: (t, 0)),
            pl.BlockSpec((D, 8), lambda t, u: (0, 0)),
            pl.BlockSpec(memory_space=pltpu.SMEM),
            pl.BlockSpec((8, 64), lambda t, u: (0, 0)),
            pl.BlockSpec((1, 64), lambda t, u: (0, 0)),
            pl.BlockSpec((1, D, H), lambda t, u: (u, 0, 0)),
            pl.BlockSpec((1, H, D), lambda t, u: (u, 0, 0)),
            pl.BlockSpec((1, D, H), lambda t, u: (u, 0, 0)),
            pl.BlockSpec((1, H, D), lambda t, u: (u, 0, 0)),
        ],
        out_specs=[
            pl.BlockSpec((NB, D), lambda t, u: (t, 0)),
            pl.BlockSpec(memory_space=pltpu.SMEM),
        ],
        out_shape=[
            jax.ShapeDtypeStruct((N, D), jnp.float32),
            jax.ShapeDtypeStruct((1, 1), jnp.float32),
        ],
        scratch_shapes=[
            pltpu.VMEM((NB, 8), jnp.float32),
            pltpu.VMEM((NB, 8), jnp.float32),
            pltpu.VMEM((1, 64), jnp.float32),
        ],
        compiler_params=pltpu.CompilerParams(
            dimension_semantics=("arbitrary", "arbitrary"),
            vmem_limit_bytes=100 * 1024 * 1024,
        ),
    )(x_flat, wrt, temp, hexT, hexsq, upper_w1, upper_w2, lower_w1, lower_w2)

    return out.reshape(B, T, D), loss.reshape(())


# speedup vs baseline: 1.3856x; 1.0087x over previous
import functools

import jax
import jax.numpy as jnp
import numpy as np
from jax.experimental import pallas as pl
from jax.experimental.pallas import tpu as pltpu

TOP_K = 4
N_EXPERTS = 64
BALANCE_COEFF = 0.01


def _hexagram_consts():
    combos = np.zeros((64, 8), dtype=np.float32)
    for e in range(64):
        for b in range(6):
            combos[e, b] = float((e >> (5 - b)) & 1)
    return combos


_HEX_NP = _hexagram_consts()


def _moe_kernel(n_total, x_ref, wrt_ref, temp_ref, hexT_ref, hexsq_ref,
                uw1_ref, uw2_ref, lw1_ref, lw2_ref,
                out_ref, loss_ref,
                a_ref, b_ref, psum_ref):
    t = pl.program_id(0)
    u = pl.program_id(1)
    n_t = pl.num_programs(0)

    @pl.when(u == 0)
    def _router():
        x = x_ref[...]
        z = jnp.dot(x, wrt_ref[...], preferred_element_type=jnp.float32)
        zsq = jnp.sum(z * z, axis=1, keepdims=True)
        cross = jnp.dot(z, hexT_ref[...], preferred_element_type=jnp.float32,
                        precision=jax.lax.Precision.HIGHEST)
        d2 = jnp.maximum(zsq - 2.0 * cross + hexsq_ref[...], 0.0)
        temp = temp_ref[0, 0]
        logits = -jnp.sqrt(d2) / temp

        n = logits.shape[0]
        iota_e = jax.lax.broadcasted_iota(jnp.int32, (n, N_EXPERTS), 1)
        cur = logits
        vals = []
        idxs = []
        for _ in range(TOP_K):
            v = jnp.max(cur, axis=1, keepdims=True)
            is_max = cur == v
            idx = jnp.min(jnp.where(is_max, iota_e, N_EXPERTS), axis=1,
                          keepdims=True)
            cur = jnp.where(iota_e == idx, -jnp.inf, cur)
            vals.append(v)
            idxs.append(idx)

        exps = [jnp.exp(v - vals[0]) for v in vals]
        denom = exps[0] + exps[1] + exps[2] + exps[3]

        iota8 = jax.lax.broadcasted_iota(jnp.int32, (n, 8), 1)
        a = jnp.zeros((n, 8), dtype=jnp.float32)
        b = jnp.zeros((n, 8), dtype=jnp.float32)
        for k in range(TOP_K):
            w = exps[k] / denom
            uid = idxs[k] // 8
            lid = idxs[k] - uid * 8
            a = a + jnp.where(iota8 == uid, w, 0.0)
            b = b + jnp.where(iota8 == lid, w, 0.0)
        a_ref[...] = a
        b_ref[...] = b

        m = jnp.max(logits, axis=1, keepdims=True)
        e = jnp.exp(logits - m)
        p = e / jnp.sum(e, axis=1, keepdims=True)
        psum = jnp.sum(p, axis=0, keepdims=True)

        @pl.when(t == 0)
        def _():
            psum_ref[...] = psum

        @pl.when(t != 0)
        def _():
            psum_ref[...] = psum_ref[...] + psum

        @pl.when(t == n_t - 1)
        def _():
            probs = psum_ref[...] * (1.0 / n_total)
            uniform = 1.0 / N_EXPERTS
            kl = jnp.sum(uniform * (jnp.log(uniform) - jnp.log(probs)))
            loss_ref[0, 0] = BALANCE_COEFF * kl

    x = x_ref[...]
    sel = jax.lax.broadcasted_iota(jnp.int32, a_ref.shape, 1) == u
    aw = jnp.sum(jnp.where(sel, a_ref[...], 0.0), axis=1, keepdims=True)
    bw = jnp.sum(jnp.where(sel, b_ref[...], 0.0), axis=1, keepdims=True)

    h_u = jnp.dot(x, uw1_ref[0], preferred_element_type=jnp.float32)
    h_u = h_u * jax.nn.sigmoid(h_u) * aw
    acc = jnp.dot(h_u, uw2_ref[0], preferred_element_type=jnp.float32)

    h_l = jnp.dot(x, lw1_ref[0], preferred_element_type=jnp.float32)
    h_l = h_l * jax.nn.sigmoid(h_l) * bw
    acc = acc + jnp.dot(h_l, lw2_ref[0], preferred_element_type=jnp.float32)

    @pl.when(u == 0)
    def _init():
        out_ref[...] = acc

    @pl.when(u != 0)
    def _acc():
        out_ref[...] = out_ref[...] + acc


@functools.partial(jax.jit, static_argnames=())
def kernel(x, W_router, log_temp, upper_w1, upper_w2, lower_w1, lower_w2):
    B, T, D = x.shape
    N = B * T
    x_flat = x.reshape(N, D)
    H = upper_w1.shape[-1]
    NB = 1024
    n_t = N // NB

    wrt = jnp.zeros((D, 8), jnp.float32).at[:, :6].set(W_router.T)
    temp = jnp.clip(jnp.exp(log_temp), 0.01, 5.0).reshape(1, 1)
    hexT = jnp.asarray(_HEX_NP.T)
    hexsq = jnp.sum(jnp.asarray(_HEX_NP) ** 2, axis=1)[None, :]

    grid = (n_t, 8)
    out, loss = pl.pallas_call(
        functools.partial(_moe_kernel, N),
        grid=grid,
        in_specs=[
            pl.BlockSpec((NB, D), lambda t, u: (t, 0)),
            pl.BlockSpec((D, 8), lambda t, u: (0, 0)),
            pl.BlockSpec(memory_space=pltpu.SMEM),
            pl.BlockSpec((8, 64), lambda t, u: (0, 0)),
            pl.BlockSpec((1, 64), lambda t, u: (0, 0)),
            pl.BlockSpec((1, D, H), lambda t, u: (u, 0, 0)),
            pl.BlockSpec((1, H, D), lambda t, u: (u, 0, 0)),
            pl.BlockSpec((1, D, H), lambda t, u: (u, 0, 0)),
            pl.BlockSpec((1, H, D), lambda t, u: (u, 0, 0)),
        ],
        out_specs=[
            pl.BlockSpec((NB, D), lambda t, u: (t, 0)),
            pl.BlockSpec(memory_space=pltpu.SMEM),
        ],
        out_shape=[
            jax.ShapeDtypeStruct((N, D), jnp.float32),
            jax.ShapeDtypeStruct((1, 1), jnp.float32),
        ],
        scratch_shapes=[
            pltpu.VMEM((NB, 8), jnp.float32),
            pltpu.VMEM((NB, 8), jnp.float32),
            pltpu.VMEM((1, 64), jnp.float32),
        ],
        compiler_params=pltpu.CompilerParams(
            dimension_semantics=("arbitrary", "arbitrary"),
        ),
    )(x_flat, wrt, temp, hexT, hexsq, upper_w1, upper_w2, lower_w1, lower_w2)

    return out.reshape(B, T, D), loss.reshape(())
